# baseline (speedup 1.0000x reference)
import functools

import jax
import jax.numpy as jnp
from jax import lax
from jax.experimental import pallas as pl
from jax.experimental.pallas import tpu as pltpu
from jax.experimental.pallas import tpu_sc as plsc

OUT_H, OUT_W = 7, 7
NBIN = OUT_H * OUT_W
SPATIAL_SCALE = 1.0 / 16.0
WIN = 8
NW = 32
HDR0 = WIN * WIN
CORN0 = HDR0 + 16
META_W = CORN0 + 16 * NBIN


def _sc_body(meta_hbm, feat_hbm, out_hbm, m0, m1, reg0, reg1, o0, o1,
             gs0, gs1, os0, os1):
    C = feat_hbm.shape[1]
    NCH = C // 16
    N = meta_hbm.shape[0]
    ms, regs, outs = [m0, m1], [reg0, reg1], [o0, o1]
    gsems, osems = [gs0, gs1], [os0, os1]
    wid = lax.axis_index("s") * 2 + lax.axis_index("c")
    count = (N - 1 - wid) // NW + 1

    def issue(t, b):
        pltpu.sync_copy(meta_hbm.at[t * NW + wid], ms[b])
        pltpu.async_copy(
            feat_hbm.at[ms[b].at[pl.ds(0, WIN * WIN)]], regs[b], gsems[b]
        )

    def gather_wait(b):
        pltpu.make_async_copy(
            feat_hbm.at[ms[b].at[pl.ds(0, WIN * WIN)]], regs[b], gsems[b]
        ).wait()

    def out_wait(b):
        pltpu.make_async_copy(outs[b], out_hbm.at[0], osems[b]).wait()

    @pl.when(count > 0)
    def _prologue():
        issue(0, 0)

    def pair_body(g, carry):
        for b in range(2):
            t = 2 * g + b
            nxt = 1 - b

            @pl.when(t + 1 < count)
            def _prefetch():
                issue(t + 1, nxt)

            @pl.when(t < count)
            def _process():
                gather_wait(b)

                @pl.when(t >= 2)
                def _drain():
                    out_wait(b)

                meta_v, reg_v, out_v = ms[b], regs[b], outs[b]
                e4 = meta_v[pl.ds(HDR0, 16)][0]

                def body4(ij, carry2):
                    cv = meta_v[pl.ds(CORN0 + ij * 16, 16)]
                    c0, c1, c2, c3, o = cv[0], cv[1], cv[2], cv[3], cv[4]
                    for c in range(NCH):
                        sl = pl.ds(c * 16, 16)
                        v = jnp.maximum(
                            jnp.maximum(reg_v[c0, sl], reg_v[c1, sl]),
                            jnp.maximum(reg_v[c2, sl], reg_v[c3, sl]),
                        )
                        out_v[o, sl] = v
                    return carry2

                def body2(ij, carry2):
                    cv = meta_v[pl.ds(CORN0 + ij * 16, 16)]
                    c0, c1, o = cv[0], cv[1], cv[4]
                    for c in range(NCH):
                        sl = pl.ds(c * 16, 16)
                        out_v[o, sl] = jnp.maximum(reg_v[c0, sl], reg_v[c1, sl])
                    return carry2

                lax.fori_loop(0, e4, body4, 0)
                lax.fori_loop(e4, NBIN, body2, 0)
                pltpu.async_copy(out_v, out_hbm.at[t * NW + wid], osems[b])

        return carry

    lax.fori_loop(0, (count + 1) // 2, pair_body, 0)

    @pl.when(count >= 1)
    def _drain0():
        out_wait(0)

    @pl.when(count >= 2)
    def _drain1():
        out_wait(1)


@jax.jit
def _roi_pool_sc(feat_rows, meta):
    R, C = feat_rows.shape
    N = meta.shape[0]
    mesh = plsc.VectorSubcoreMesh(core_axis_name="c", subcore_axis_name="s")
    f = functools.partial(
        pl.kernel,
        mesh=mesh,
        compiler_params=pltpu.CompilerParams(
            needs_layout_passes=False, use_tc_tiling_on_sc=False
        ),
        out_type=jax.ShapeDtypeStruct((N, NBIN, C), jnp.float32),
        scratch_types=[
            pltpu.VMEM((META_W,), jnp.int32),
            pltpu.VMEM((META_W,), jnp.int32),
            pltpu.VMEM((WIN * WIN, C), jnp.float32),
            pltpu.VMEM((WIN * WIN, C), jnp.float32),
            pltpu.VMEM((NBIN, C), jnp.float32),
            pltpu.VMEM((NBIN, C), jnp.float32),
            pltpu.SemaphoreType.DMA,
            pltpu.SemaphoreType.DMA,
            pltpu.SemaphoreType.DMA,
            pltpu.SemaphoreType.DMA,
        ],
    )(_sc_body)
    return f(meta, feat_rows)


def kernel(features, rois, roi_indices):
    B, C, H, W = features.shape
    N = rois.shape[0]
    rois_i = (rois * SPATIAL_SCALE).astype(jnp.int32)
    img = roi_indices.astype(jnp.int32)
    hx, wy = rois_i[:, 0], rois_i[:, 1]
    lh = rois_i[:, 2] - hx
    lw = rois_i[:, 3] - wy
    hs = jnp.clip(hx, 0, H - WIN)
    ws = jnp.clip(wy, 0, W - WIN)

    p = jnp.arange(WIN * WIN, dtype=jnp.int32)
    idx_rows = (img * (H * W))[:, None] + (hs[:, None] + p[None, :] // WIN) * W \
        + (ws[:, None] + p[None, :] % WIN)

    def bounds(l, start, wstart, n_out):
        i = jnp.arange(n_out, dtype=jnp.int32)[None, :]
        r0 = (i * l[:, None]) // n_out
        r1m = -(((-(i + 1)) * l[:, None]) // n_out) - 1
        r1m = jnp.maximum(r1m, r0)
        off = (start - wstart)[:, None]
        return jnp.clip(r0 + off, 0, WIN - 1), jnp.clip(r1m + off, 0, WIN - 1)

    x0, x1 = bounds(lh, hx, hs, OUT_H)
    y0, y1 = bounds(lw, wy, ws, OUT_W)
    corners = jnp.stack(
        [
            x0[:, :, None, None] * WIN + y0[:, None, :, None],
            x0[:, :, None, None] * WIN + y1[:, None, :, None],
            x1[:, :, None, None] * WIN + y0[:, None, :, None],
            x1[:, :, None, None] * WIN + y1[:, None, :, None],
        ],
        axis=3,
    ).reshape(N, NBIN, 4)

    wxb = jnp.broadcast_to((x1 > x0)[:, :, None], (N, OUT_H, OUT_W))
    wyb = jnp.broadcast_to((y1 > y0)[:, None, :], (N, OUT_H, OUT_W))
    is4 = (wxb & wyb).reshape(N, NBIN)
    k4 = is4.sum(axis=1).astype(jnp.int32)
    c1sel = jnp.where(wyb.reshape(N, NBIN), corners[:, :, 1], corners[:, :, 2])
    ijo = jnp.broadcast_to(jnp.arange(NBIN, dtype=jnp.int32)[None], (N, NBIN))
    all5 = jnp.stack(
        [corners[:, :, 0], c1sel, corners[:, :, 2], corners[:, :, 3], ijo],
        axis=2,
    )
    order = jnp.argsort(jnp.where(is4, 0, 1).astype(jnp.int32), axis=1)
    ordered = jnp.take_along_axis(all5, order[:, :, None], axis=1)
    cells = jnp.zeros((N, NBIN, 16), jnp.int32).at[:, :, :5].set(ordered)
    hdr = jnp.zeros((N, 16), jnp.int32).at[:, 0].set(k4)
    meta = jnp.concatenate(
        [idx_rows, hdr, cells.reshape(N, 16 * NBIN)], axis=1
    )

    feat_rows = features.transpose(0, 2, 3, 1).reshape(B * H * W, C)
    out = _roi_pool_sc(feat_rows, meta)
    return out.transpose(0, 2, 1).reshape(N, C, OUT_H, OUT_W)


# speedup vs baseline: 1.6245x; 1.6245x over previous
import functools

import jax
import jax.numpy as jnp
from jax import lax
from jax.experimental import pallas as pl
from jax.experimental.pallas import tpu as pltpu
from jax.experimental.pallas import tpu_sc as plsc

OUT_H, OUT_W = 7, 7
NBIN = OUT_H * OUT_W
SPATIAL_SCALE = 1.0 / 16.0
WIN = 8
NW = 32
HDR0 = WIN * WIN
XG0 = HDR0 + 16
YG0 = XG0 + 16 * OUT_H
META_W = YG0 + 16 * OUT_W


def _sc_body(meta_hbm, feat_hbm, out_hbm, m0, m1, reg0, reg1, o0, o1,
             gs0, gs1, os0, os1):
    C = feat_hbm.shape[1]
    NCH = C // 16
    N = meta_hbm.shape[0]
    ms, regs, outs = [m0, m1], [reg0, reg1], [o0, o1]
    gsems, osems = [gs0, gs1], [os0, os1]
    wid = lax.axis_index("s") * 2 + lax.axis_index("c")
    count = (N - 1 - wid) // NW + 1

    def issue(t, b):
        pltpu.sync_copy(meta_hbm.at[t * NW + wid], ms[b])
        pltpu.async_copy(
            feat_hbm.at[ms[b].at[pl.ds(0, WIN * WIN)]], regs[b], gsems[b]
        )

    def gather_wait(b):
        pltpu.make_async_copy(
            feat_hbm.at[ms[b].at[pl.ds(0, WIN * WIN)]], regs[b], gsems[b]
        ).wait()

    def out_wait(b):
        pltpu.make_async_copy(outs[b], out_hbm.at[0], osems[b]).wait()

    @pl.when(count > 0)
    def _prologue():
        issue(0, 0)

    def pair_body(g, carry):
        for b in range(2):
            t = 2 * g + b
            nxt = 1 - b

            @pl.when(t + 1 < count)
            def _prefetch():
                issue(t + 1, nxt)

            @pl.when(t < count)
            def _process():
                gather_wait(b)

                @pl.when(t >= 2)
                def _drain():
                    out_wait(b)

                meta_v, reg_v, out_v = ms[b], regs[b], outs[b]
                hv = meta_v[pl.ds(HDR0, 16)]
                kx, ky = hv[0], hv[1]

                def xbin(xb):
                    xv = meta_v[pl.ds(XG0 + xb * 16, 16)]
                    return xv[0], xv[1], xv[2]

                def ybin(yb):
                    yv = meta_v[pl.ds(YG0 + yb * 16, 16)]
                    return yv[0], yv[1], yv[2]

                def max2(c0, c1, o):
                    for c in range(NCH):
                        sl = pl.ds(c * 16, 16)
                        out_v[o, sl] = jnp.maximum(reg_v[c0, sl], reg_v[c1, sl])

                def a_x(xb, carry2):
                    sx0, sx1, sox = xbin(xb)

                    def a_y(yb, c2_):
                        sy0, sy1, soy = ybin(yb)
                        o = sox + soy
                        for c in range(NCH):
                            sl = pl.ds(c * 16, 16)
                            v = jnp.maximum(
                                jnp.maximum(
                                    reg_v[sx0 + sy0, sl], reg_v[sx0 + sy1, sl]
                                ),
                                jnp.maximum(
                                    reg_v[sx1 + sy0, sl], reg_v[sx1 + sy1, sl]
                                ),
                            )
                            out_v[o, sl] = v
                        return c2_

                    lax.fori_loop(0, ky, a_y, 0)
                    return carry2

                def b_x(xb, carry2):
                    sx0, sx1, sox = xbin(xb)

                    def b_y(yb, c2_):
                        sy0, _, soy = ybin(yb)
                        max2(sx0 + sy0, sx1 + sy0, sox + soy)
                        return c2_

                    lax.fori_loop(ky, OUT_W, b_y, 0)
                    return carry2

                def c_x(xb, carry2):
                    sx0, _, sox = xbin(xb)

                    def c_y(yb, c2_):
                        sy0, sy1, soy = ybin(yb)
                        max2(sx0 + sy0, sx0 + sy1, sox + soy)
                        return c2_

                    lax.fori_loop(0, OUT_W, c_y, 0)
                    return carry2

                lax.fori_loop(0, kx, a_x, 0)
                lax.fori_loop(0, kx, b_x, 0)
                lax.fori_loop(kx, OUT_H, c_x, 0)
                pltpu.async_copy(out_v, out_hbm.at[t * NW + wid], osems[b])

        return carry

    lax.fori_loop(0, (count + 1) // 2, pair_body, 0)

    @pl.when(count >= 1)
    def _drain0():
        out_wait(0)

    @pl.when(count >= 2)
    def _drain1():
        out_wait(1)


@jax.jit
def _roi_pool_sc(feat_rows, meta):
    R, C = feat_rows.shape
    N = meta.shape[0]
    mesh = plsc.VectorSubcoreMesh(core_axis_name="c", subcore_axis_name="s")
    f = functools.partial(
        pl.kernel,
        mesh=mesh,
        compiler_params=pltpu.CompilerParams(
            needs_layout_passes=False, use_tc_tiling_on_sc=False
        ),
        out_type=jax.ShapeDtypeStruct((N, NBIN, C), jnp.float32),
        scratch_types=[
            pltpu.VMEM((META_W,), jnp.int32),
            pltpu.VMEM((META_W,), jnp.int32),
            pltpu.VMEM((WIN * WIN, C), jnp.float32),
            pltpu.VMEM((WIN * WIN, C), jnp.float32),
            pltpu.VMEM((NBIN, C), jnp.float32),
            pltpu.VMEM((NBIN, C), jnp.float32),
            pltpu.SemaphoreType.DMA,
            pltpu.SemaphoreType.DMA,
            pltpu.SemaphoreType.DMA,
            pltpu.SemaphoreType.DMA,
        ],
    )(_sc_body)
    return f(meta, feat_rows)


def kernel(features, rois, roi_indices):
    B, C, H, W = features.shape
    N = rois.shape[0]
    rois_i = (rois * SPATIAL_SCALE).astype(jnp.int32)
    img = roi_indices.astype(jnp.int32)
    hx, wy = rois_i[:, 0], rois_i[:, 1]
    lh = rois_i[:, 2] - hx
    lw = rois_i[:, 3] - wy
    hs = jnp.clip(hx, 0, H - WIN)
    ws = jnp.clip(wy, 0, W - WIN)

    p = jnp.arange(WIN * WIN, dtype=jnp.int32)
    idx_rows = (img * (H * W))[:, None] + (hs[:, None] + p[None, :] // WIN) * W \
        + (ws[:, None] + p[None, :] % WIN)

    def bounds(l, start, wstart, n_out):
        i = jnp.arange(n_out, dtype=jnp.int32)[None, :]
        r0 = (i * l[:, None]) // n_out
        r1m = -(((-(i + 1)) * l[:, None]) // n_out) - 1
        r1m = jnp.maximum(r1m, r0)
        off = (start - wstart)[:, None]
        return jnp.clip(r0 + off, 0, WIN - 1), jnp.clip(r1m + off, 0, WIN - 1)

    x0, x1 = bounds(lh, hx, hs, OUT_H)
    y0, y1 = bounds(lw, wy, ws, OUT_W)

    def partition(a0, a1, mul, omul, n_out):
        wide = a1 > a0
        k = wide.sum(axis=1).astype(jnp.int32)
        order = jnp.argsort(jnp.where(wide, 0, 1).astype(jnp.int32), axis=1)
        oidx = jnp.broadcast_to(
            jnp.arange(n_out, dtype=jnp.int32)[None], a0.shape
        )
        tri = jnp.stack([a0 * mul, a1 * mul, oidx * omul], axis=2)
        tri = jnp.take_along_axis(tri, order[:, :, None], axis=1)
        g = jnp.zeros((N, n_out, 16), jnp.int32).at[:, :, :3].set(tri)
        return k, g.reshape(N, 16 * n_out)

    kx, xg = partition(x0, x1, WIN, OUT_W, OUT_H)
    ky, yg = partition(y0, y1, 1, 1, OUT_W)
    hdr = jnp.zeros((N, 16), jnp.int32).at[:, 0].set(kx).at[:, 1].set(ky)
    meta = jnp.concatenate([idx_rows, hdr, xg, yg], axis=1)

    feat_rows = features.transpose(0, 2, 3, 1).reshape(B * H * W, C)
    out = _roi_pool_sc(feat_rows, meta)
    return out.transpose(0, 2, 1).reshape(N, C, OUT_H, OUT_W)


# speedup vs baseline: 1.7091x; 1.0521x over previous
import functools

import jax
import jax.numpy as jnp
from jax import lax
from jax.experimental import pallas as pl
from jax.experimental.pallas import tpu as pltpu
from jax.experimental.pallas import tpu_sc as plsc

OUT_H, OUT_W = 7, 7
NBIN = OUT_H * OUT_W
SPATIAL_SCALE = 1.0 / 16.0
WIN = 8
NW = 32
CORN0 = WIN * WIN
META_W = CORN0 + 16 * (NBIN + 1)


def _sc_body(meta_hbm, feat_hbm, out_hbm, m0, m1, reg0, reg1, o0, o1,
             gs0, gs1, os0, os1):
    C = feat_hbm.shape[1]
    NCH = C // 16
    N = meta_hbm.shape[0]
    ms, regs, outs = [m0, m1], [reg0, reg1], [o0, o1]
    gsems, osems = [gs0, gs1], [os0, os1]
    wid = lax.axis_index("s") * 2 + lax.axis_index("c")
    count = (N - 1 - wid) // NW + 1

    def issue(t, b):
        pltpu.sync_copy(meta_hbm.at[t * NW + wid], ms[b])
        pltpu.async_copy(
            feat_hbm.at[ms[b].at[pl.ds(0, WIN * WIN)]], regs[b], gsems[b]
        )

    def gather_wait(b):
        pltpu.make_async_copy(
            feat_hbm.at[ms[b].at[pl.ds(0, WIN * WIN)]], regs[b], gsems[b]
        ).wait()

    def out_wait(b):
        pltpu.make_async_copy(outs[b], out_hbm.at[0], osems[b]).wait()

    @pl.when(count > 0)
    def _prologue():
        issue(0, 0)

    def pair_body(g, carry):
        for b in range(2):
            t = 2 * g + b
            nxt = 1 - b

            @pl.when(t + 1 < count)
            def _prefetch():
                issue(t + 1, nxt)

            @pl.when(t < count)
            def _process():
                gather_wait(b)

                @pl.when(t >= 2)
                def _drain():
                    out_wait(b)

                meta_v, reg_v, out_v = ms[b], regs[b], outs[b]

                def corners_of(ij):
                    cv = meta_v[pl.ds(CORN0 + ij * 16, 16)]
                    return cv[0], cv[1], cv[2], cv[3]

                def bin_body(ij, carry2):
                    c0, c1, c2, c3 = carry2
                    nxt_c = corners_of(ij + 1)
                    for c in range(NCH):
                        sl = pl.ds(c * 16, 16)
                        v = jnp.maximum(
                            jnp.maximum(reg_v[c0, sl], reg_v[c1, sl]),
                            jnp.maximum(reg_v[c2, sl], reg_v[c3, sl]),
                        )
                        out_v[ij, sl] = v
                    return nxt_c

                lax.fori_loop(0, NBIN, bin_body, corners_of(0))
                pltpu.async_copy(out_v, out_hbm.at[t * NW + wid], osems[b])

        return carry

    lax.fori_loop(0, (count + 1) // 2, pair_body, 0)

    @pl.when(count >= 1)
    def _drain0():
        out_wait(0)

    @pl.when(count >= 2)
    def _drain1():
        out_wait(1)


@jax.jit
def _roi_pool_sc(feat_rows, meta):
    R, C = feat_rows.shape
    N = meta.shape[0]
    mesh = plsc.VectorSubcoreMesh(core_axis_name="c", subcore_axis_name="s")
    f = functools.partial(
        pl.kernel,
        mesh=mesh,
        compiler_params=pltpu.CompilerParams(
            needs_layout_passes=False, use_tc_tiling_on_sc=False
        ),
        out_type=jax.ShapeDtypeStruct((N, NBIN, C), jnp.float32),
        scratch_types=[
            pltpu.VMEM((META_W,), jnp.int32),
            pltpu.VMEM((META_W,), jnp.int32),
            pltpu.VMEM((WIN * WIN, C), jnp.float32),
            pltpu.VMEM((WIN * WIN, C), jnp.float32),
            pltpu.VMEM((NBIN, C), jnp.float32),
            pltpu.VMEM((NBIN, C), jnp.float32),
            pltpu.SemaphoreType.DMA,
            pltpu.SemaphoreType.DMA,
            pltpu.SemaphoreType.DMA,
            pltpu.SemaphoreType.DMA,
        ],
    )(_sc_body)
    return f(meta, feat_rows)


def kernel(features, rois, roi_indices):
    B, C, H, W = features.shape
    N = rois.shape[0]
    rois_i = (rois * SPATIAL_SCALE).astype(jnp.int32)
    img = roi_indices.astype(jnp.int32)
    hx, wy = rois_i[:, 0], rois_i[:, 1]
    lh = rois_i[:, 2] - hx
    lw = rois_i[:, 3] - wy
    hs = jnp.clip(hx, 0, H - WIN)
    ws = jnp.clip(wy, 0, W - WIN)

    p = jnp.arange(WIN * WIN, dtype=jnp.int32)
    idx_rows = (img * (H * W))[:, None] + (hs[:, None] + p[None, :] // WIN) * W \
        + (ws[:, None] + p[None, :] % WIN)

    def bounds(l, start, wstart, n_out):
        i = jnp.arange(n_out, dtype=jnp.int32)[None, :]
        r0 = (i * l[:, None]) // n_out
        r1m = -(((-(i + 1)) * l[:, None]) // n_out) - 1
        r1m = jnp.maximum(r1m, r0)
        off = (start - wstart)[:, None]
        return jnp.clip(r0 + off, 0, WIN - 1), jnp.clip(r1m + off, 0, WIN - 1)

    x0, x1 = bounds(lh, hx, hs, OUT_H)
    y0, y1 = bounds(lw, wy, ws, OUT_W)
    corners = jnp.stack(
        [
            x0[:, :, None, None] * WIN + y0[:, None, :, None],
            x0[:, :, None, None] * WIN + y1[:, None, :, None],
            x1[:, :, None, None] * WIN + y0[:, None, :, None],
            x1[:, :, None, None] * WIN + y1[:, None, :, None],
        ],
        axis=3,
    ).reshape(N, NBIN, 4)
    cells = jnp.zeros((N, NBIN + 1, 16), jnp.int32).at[:, :NBIN, :4].set(corners)
    meta = jnp.concatenate([idx_rows, cells.reshape(N, 16 * (NBIN + 1))], axis=1)

    feat_rows = features.transpose(0, 2, 3, 1).reshape(B * H * W, C)
    out = _roi_pool_sc(feat_rows, meta)
    return out.transpose(0, 2, 1).reshape(N, C, OUT_H, OUT_W)
